# Initial kernel scaffold; baseline (speedup 1.0000x reference)
import jax
import jax.numpy as jnp
from jax.experimental import pallas as pl


def kernel(x, edge_index, batch, W1, b1, W2, b2, W3, b3, Wp, bp, Wl1, bl1, Wl2, bl2, Wc, bc):
    raise NotImplementedError("write your pallas kernel here")


# speedup vs baseline: 9.5449x; 9.5449x over previous
import functools

import jax
import jax.numpy as jnp
from jax import lax
from jax.experimental import pallas as pl
from jax.experimental.pallas import tpu as pltpu
from jax.experimental.pallas import tpu_sc as plsc

N = 10000
NPAD = 10240
E = 320000
D = 128
K = 5000
NC = 2
NS = 16
NW = NC * NS
EPT = E // NW
B = 80
NB = EPT // B
RPT = NPAD // NS
R = 1000
GRID = N // R


@functools.lru_cache(maxsize=None)
def _sc_kernels():
    mesh = plsc.VectorSubcoreMesh(core_axis_name="c", subcore_axis_name="s",
                                  num_cores=NC, num_subcores=NS)

    @functools.partial(
        pl.kernel,
        out_type=jax.ShapeDtypeStruct((NC, NPAD, D), jnp.float32),
        mesh=mesh,
        scratch_types=[
            pltpu.VMEM((B,), jnp.int32),
            pltpu.VMEM((B,), jnp.int32),
            pltpu.VMEM((B, D), jnp.float32),
            pltpu.VMEM_SHARED((NPAD, D), jnp.float32),
            pltpu.SemaphoreType.DMA,
        ],
    )
    def _sc_spmm_rows(hs_hbm, src_hbm, dst_hbm, zeros_hbm, out_hbm,
                      sidx, didx, rows, acc, sem):
        c = lax.axis_index("c")
        s = lax.axis_index("s")
        pltpu.sync_copy(zeros_hbm, rows)
        for k in range(RPT // B):
            pltpu.sync_copy(rows, acc.at[pl.ds(s * RPT + k * B, B)])
        plsc.subcore_barrier()
        base = (c * NS + s) * EPT

        def body(b, carry):
            off = base + b * B
            pltpu.sync_copy(src_hbm.at[pl.ds(off, B)], sidx)
            pltpu.sync_copy(dst_hbm.at[pl.ds(off, B)], didx)
            pltpu.async_copy(hs_hbm.at[sidx], rows, sem).wait()
            pltpu.sync_copy(rows, acc.at[didx], add=True)
            return carry

        lax.fori_loop(0, NB, body, 0)
        plsc.subcore_barrier()
        pltpu.sync_copy(acc.at[pl.ds(s * RPT, RPT)],
                        out_hbm.at[c, pl.ds(s * RPT, RPT)])

    @functools.partial(
        pl.kernel,
        out_type=jax.ShapeDtypeStruct((NC, NPAD), jnp.float32),
        mesh=mesh,
        scratch_types=[
            pltpu.VMEM((B,), jnp.int32),
            pltpu.VMEM((B,), jnp.int32),
            pltpu.VMEM((B,), jnp.float32),
            pltpu.VMEM_SHARED((NPAD,), jnp.float32),
            pltpu.SemaphoreType.DMA,
        ],
    )
    def _sc_spmm_scalar(tab_hbm, src_hbm, dst_hbm, zeros_hbm, out_hbm,
                        sidx, didx, vals, acc, sem):
        c = lax.axis_index("c")
        s = lax.axis_index("s")
        pltpu.sync_copy(zeros_hbm, vals)
        for k in range(RPT // B):
            pltpu.sync_copy(vals, acc.at[pl.ds(s * RPT + k * B, B)])
        plsc.subcore_barrier()
        base = (c * NS + s) * EPT

        def body(b, carry):
            off = base + b * B
            pltpu.sync_copy(src_hbm.at[pl.ds(off, B)], sidx)
            pltpu.sync_copy(dst_hbm.at[pl.ds(off, B)], didx)
            pltpu.async_copy(tab_hbm.at[sidx], vals, sem).wait()
            pltpu.sync_copy(vals, acc.at[didx], add=True)
            return carry

        lax.fori_loop(0, NB, body, 0)
        plsc.subcore_barrier()
        pltpu.sync_copy(acc.at[pl.ds(s * RPT, RPT)],
                        out_hbm.at[c, pl.ds(s * RPT, RPT)])

    return _sc_spmm_rows, _sc_spmm_scalar


def _pre_body(cnt0, cnt1, x, W1, dinv_o, hs1_o):
    dv = lax.rsqrt(1.0 + cnt0[...] + cnt1[...])
    dinv_o[...] = dv
    hs1_o[...] = dv * jnp.dot(x[...], W1[...], preferred_element_type=jnp.float32)


def _tc_pre(cnt0, cnt1, x, W1):
    return pl.pallas_call(
        _pre_body,
        grid=(GRID,),
        in_specs=[
            pl.BlockSpec((R, 1), lambda i: (i, 0)),
            pl.BlockSpec((R, 1), lambda i: (i, 0)),
            pl.BlockSpec((R, D), lambda i: (i, 0)),
            pl.BlockSpec((D, D), lambda i: (0, 0)),
        ],
        out_specs=[
            pl.BlockSpec((R, 1), lambda i: (i, 0)),
            pl.BlockSpec((R, D), lambda i: (i, 0)),
        ],
        out_shape=[
            jax.ShapeDtypeStruct((N, 1), jnp.float32),
            jax.ShapeDtypeStruct((N, D), jnp.float32),
        ],
    )(cnt0, cnt1, x, W1)


def _layer_body(p0, p1, hs, dinv, b, Wn, x_o, hsn_o):
    dv = dinv[...]
    xo = jnp.maximum(dv * (p0[...] + p1[...] + hs[...]) + b[...], 0.0)
    x_o[...] = xo
    hsn_o[...] = dv * jnp.dot(xo, Wn[...], preferred_element_type=jnp.float32)


def _tc_layer(p0, p1, hs, dinv, b, Wn):
    return pl.pallas_call(
        _layer_body,
        grid=(GRID,),
        in_specs=[
            pl.BlockSpec((R, D), lambda i: (i, 0)),
            pl.BlockSpec((R, D), lambda i: (i, 0)),
            pl.BlockSpec((R, D), lambda i: (i, 0)),
            pl.BlockSpec((R, 1), lambda i: (i, 0)),
            pl.BlockSpec((1, D), lambda i: (0, 0)),
            pl.BlockSpec((D, D), lambda i: (0, 0)),
        ],
        out_specs=[
            pl.BlockSpec((R, D), lambda i: (i, 0)),
            pl.BlockSpec((R, D), lambda i: (i, 0)),
        ],
        out_shape=[
            jax.ShapeDtypeStruct((N, D), jnp.float32),
            jax.ShapeDtypeStruct((N, D), jnp.float32),
        ],
    )(p0, p1, hs, dinv, b, Wn)


def _layer3_body(p0, p1, hs, dinv, b, x1, x2, Wp, x3_o, vs_o):
    dv = dinv[...]
    x3 = jnp.maximum(dv * (p0[...] + p1[...] + hs[...]) + b[...], 0.0)
    x3_o[...] = x3
    Wpv = Wp[...]
    v = (jnp.dot(x1[...], Wpv[0:D], preferred_element_type=jnp.float32)
         + jnp.dot(x2[...], Wpv[D:2 * D], preferred_element_type=jnp.float32)
         + jnp.dot(x3, Wpv[2 * D:3 * D], preferred_element_type=jnp.float32))
    vs_o[...] = dv * v


def _tc_layer3(p0, p1, hs, dinv, b, x1, x2, Wp):
    return pl.pallas_call(
        _layer3_body,
        grid=(GRID,),
        in_specs=[
            pl.BlockSpec((R, D), lambda i: (i, 0)),
            pl.BlockSpec((R, D), lambda i: (i, 0)),
            pl.BlockSpec((R, D), lambda i: (i, 0)),
            pl.BlockSpec((R, 1), lambda i: (i, 0)),
            pl.BlockSpec((1, D), lambda i: (0, 0)),
            pl.BlockSpec((R, D), lambda i: (i, 0)),
            pl.BlockSpec((R, D), lambda i: (i, 0)),
            pl.BlockSpec((3 * D, 1), lambda i: (0, 0)),
        ],
        out_specs=[
            pl.BlockSpec((R, D), lambda i: (i, 0)),
            pl.BlockSpec((R, 1), lambda i: (i, 0)),
        ],
        out_shape=[
            jax.ShapeDtypeStruct((N, D), jnp.float32),
            jax.ShapeDtypeStruct((N, 1), jnp.float32),
        ],
    )(p0, p1, hs, dinv, b, x1, x2, Wp)


_ROWS = 8
_COLS = NPAD // _ROWS


def _select_body(a0, a1, vs, dinv, bp, w_o, nf_o):
    score = jnp.tanh(dinv[...] * (a0[...] + a1[...] + vs[...]) + bp[0, 0])
    rr = lax.broadcasted_iota(jnp.int32, (_ROWS, _COLS), 0)
    cc = lax.broadcasted_iota(jnp.int32, (_ROWS, _COLS), 1)
    fi = rr * _COLS + cc
    valid = fi < N
    bits = lax.bitcast_convert_type(score, jnp.int32)
    ks = bits ^ jnp.where(bits < 0, jnp.int32(0x7FFFFFFF), jnp.int32(0))
    ks = jnp.where(valid, ks, jnp.int32(-2147483648))

    def cnt_ge(v):
        return jnp.sum((ks >= v).astype(jnp.int32))

    nn_ok = cnt_ge(jnp.int32(0)) >= K
    lo = jnp.where(nn_ok, jnp.int32(0), jnp.int32(-2147483648))
    hi = jnp.where(nn_ok, jnp.int32(2147483647), jnp.int32(-1))

    def body(i, lh):
        l, h = lh
        d = h - l
        mid = l + d // 2 + (d & 1)
        ok = cnt_ge(mid) >= K
        return (jnp.where(ok, mid, l), jnp.where(ok, h, mid - 1))

    T, _ = lax.fori_loop(0, 32, body, (lo, hi))
    n_gt = jnp.sum((ks > T).astype(jnp.int32))
    m = K - n_gt
    tie = ks == T

    def cnt_tie_lt(c):
        return jnp.sum((tie & (fi < c)).astype(jnp.int32))

    def body2(i, lh):
        l, h = lh
        mid = (l + h) // 2
        ok = cnt_tie_lt(mid) >= m
        return (jnp.where(ok, l, mid + 1), jnp.where(ok, mid, h))

    _, c = lax.fori_loop(0, 15, body2, (jnp.int32(0), jnp.int32(N)))
    sel = (ks > T) | (tie & (fi < c))
    w_o[...] = jnp.where(sel, score, 0.0)
    nf_o[...] = jnp.where(sel, 0.0, -jnp.inf)


def _tc_select(a0, a1, vs, dinv, bp):
    return pl.pallas_call(
        _select_body,
        out_shape=[
            jax.ShapeDtypeStruct((_ROWS, _COLS), jnp.float32),
            jax.ShapeDtypeStruct((_ROWS, _COLS), jnp.float32),
        ],
    )(a0, a1, vs, dinv, bp)


def _pool_body(x1, x2, x3, w, nf, Wl1, bl1, Wl2, bl2, Wc, bc, out_o,
               smax, ssum):
    i = pl.program_id(0)

    @pl.when(i == 0)
    def _():
        smax[...] = jnp.full((1, 3 * D), -jnp.inf, jnp.float32)
        ssum[...] = jnp.zeros((1, 3 * D), jnp.float32)

    wv = w[...]
    nfv = nf[...]
    for part, xref in enumerate((x1, x2, x3)):
        xv = xref[...]
        pm = jnp.max(wv * xv + nfv, axis=0)[None, :]
        ps = jnp.sum(wv * xv, axis=0)[None, :]
        sl = pl.ds(part * D, D)
        smax[:, sl] = jnp.maximum(smax[:, sl], pm)
        ssum[:, sl] = ssum[:, sl] + ps

    @pl.when(i == GRID - 1)
    def _():
        g = jnp.concatenate([smax[...], ssum[...] * (1.0 / K)], axis=1)
        h = jnp.maximum(jnp.dot(g, Wl1[...], preferred_element_type=jnp.float32)
                        + bl1[...], 0.0)
        h = jnp.maximum(jnp.dot(h, Wl2[...], preferred_element_type=jnp.float32)
                        + bl2[...], 0.0)
        lg = (jnp.dot(h, Wc[...], preferred_element_type=jnp.float32) + bc[...])
        mx = jnp.max(lg)
        out_o[...] = lg - (mx + jnp.log(jnp.sum(jnp.exp(lg - mx))))


def _tc_pool(x1, x2, x3, w, nf, Wl1, bl1, Wl2, bl2, Wc, bc):
    H2 = Wl2.shape[1]
    NF = Wc.shape[1]
    return pl.pallas_call(
        _pool_body,
        grid=(GRID,),
        in_specs=[
            pl.BlockSpec((R, D), lambda i: (i, 0)),
            pl.BlockSpec((R, D), lambda i: (i, 0)),
            pl.BlockSpec((R, D), lambda i: (i, 0)),
            pl.BlockSpec((R, 1), lambda i: (i, 0)),
            pl.BlockSpec((R, 1), lambda i: (i, 0)),
            pl.BlockSpec((6 * D, D), lambda i: (0, 0)),
            pl.BlockSpec((1, D), lambda i: (0, 0)),
            pl.BlockSpec((D, H2), lambda i: (0, 0)),
            pl.BlockSpec((1, H2), lambda i: (0, 0)),
            pl.BlockSpec((H2, NF), lambda i: (0, 0)),
            pl.BlockSpec((1, NF), lambda i: (0, 0)),
        ],
        out_specs=pl.BlockSpec((1, NF), lambda i: (0, 0)),
        out_shape=jax.ShapeDtypeStruct((1, NF), jnp.float32),
        scratch_shapes=[
            pltpu.VMEM((1, 3 * D), jnp.float32),
            pltpu.VMEM((1, 3 * D), jnp.float32),
        ],
    )(x1, x2, x3, w, nf, Wl1, bl1, Wl2, bl2, Wc, bc)


def _pad_r(v):
    return jnp.pad(v[:, 0], (0, NPAD - N)).reshape(_ROWS, _COLS)


def kernel(x, edge_index, batch, W1, b1, W2, b2, W3, b3, Wp, bp,
           Wl1, bl1, Wl2, bl2, Wc, bc):
    _sc_spmm_rows, _sc_spmm_scalar = _sc_kernels()
    src = edge_index[0]
    dst = edge_index[1]
    zeros_row = jnp.zeros((B, D), jnp.float32)
    zeros_s = jnp.zeros((B,), jnp.float32)
    ones_n = jnp.ones((N,), jnp.float32)

    cntP = _sc_spmm_scalar(ones_n, src, dst, zeros_s)
    cnt0 = cntP[0, :N, None]
    cnt1 = cntP[1, :N, None]

    dinv, hs1 = _tc_pre(cnt0, cnt1, x, W1)

    p = _sc_spmm_rows(hs1, src, dst, zeros_row)
    x1, hs2 = _tc_layer(p[0, :N], p[1, :N], hs1, dinv, b1[None, :], W2)

    p = _sc_spmm_rows(hs2, src, dst, zeros_row)
    x2, hs3 = _tc_layer(p[0, :N], p[1, :N], hs2, dinv, b2[None, :], W3)

    p = _sc_spmm_rows(hs3, src, dst, zeros_row)
    x3, vs = _tc_layer3(p[0, :N], p[1, :N], hs3, dinv, b3[None, :], x1, x2, Wp)

    aP = _sc_spmm_scalar(vs[:, 0], src, dst, zeros_s)
    a0 = aP[0].reshape(_ROWS, _COLS)
    a1 = aP[1].reshape(_ROWS, _COLS)

    w, nf = _tc_select(a0, a1, _pad_r(vs), _pad_r(dinv), bp[None, :])
    w2 = w.reshape(NPAD)[:N, None]
    nf2 = nf.reshape(NPAD)[:N, None]

    return _tc_pool(x1, x2, x3, w2, nf2, Wl1, bl1[None, :], Wl2, bl2[None, :],
                    Wc, bc[None, :])
